# baseline (speedup 1.0000x reference)
import functools

import jax
import jax.numpy as jnp
from jax import lax
from jax.experimental import pallas as pl
from jax.experimental.pallas import tpu as pltpu
from jax.experimental.pallas import tpu_sc as plsc

B, S, NIN, NH, K = 16384, 8, 128, 64, 10
INV_T = 5.0
CW = 2 * NH


TB = 256
GRID = B // TB


def _stage_a_body(bf_ref, bam_ref, bfm_ref, ba_ref, sn_ref,
                  wn_ref, wc_ref, bn_ref, bc_ref, an_ref, ac_ref,
                  c_ref, tgt_ref, npos_ref, spos_ref):
    x1 = jnp.dot(bf_ref[...], wn_ref[...],
                 preferred_element_type=jnp.float32).reshape(TB, S, NH)
    x2 = jnp.dot(bfm_ref[...], wc_ref[...],
                 preferred_element_type=jnp.float32).reshape(TB, S, NH)
    x12 = jnp.concatenate([x1, x2], axis=2)

    acat = jnp.concatenate([bam_ref[...], ba_ref[...]], axis=2)
    acat2 = acat.reshape(TB * S, 2 * S)
    col = lax.broadcasted_iota(jnp.int32, (2 * S, S * CW), 1)
    row = lax.broadcasted_iota(jnp.int32, (2 * S, S * CW), 0)
    jcol = col // CW
    hcol = col % CW
    emat = (((row == jcol) & (hcol < NH))
            | ((row == jcol + S) & (hcol >= NH))).astype(jnp.float32)
    aexp = jnp.dot(acat2, emat,
                   preferred_element_type=jnp.float32)

    acc = jnp.zeros((TB * S, CW), jnp.float32)
    for j in range(S):
        xb = jnp.broadcast_to(x12[:, j:j + 1, :], (TB, S, CW))
        acc = acc + aexp[:, j * CW:(j + 1) * CW] * xb.reshape(TB * S, CW)

    b12 = jnp.concatenate([bn_ref[...], bc_ref[...]], axis=1)
    lane = lax.broadcasted_iota(jnp.int32, (1, CW), 1)
    slope = jnp.where(lane < NH, an_ref[0, 0], ac_ref[0, 0])
    h12 = acc + b12
    h12 = jnp.where(h12 >= 0, h12, slope * h12)
    h12 = h12.reshape(TB, S, CW)

    h1 = h12[:, :, :NH]
    h2 = h12[:, :, NH:]
    ss = jnp.sum(h1 * h1, axis=2, keepdims=True)
    inv = lax.rsqrt(jnp.maximum(ss, 1e-24))
    h1n = h1 * inv
    tgt = h1n[:, S - 1, :]
    tgt_ref[...] = tgt

    sn = sn_ref[...]
    onehot = (lax.broadcasted_iota(jnp.int32, (TB, S), 1) == sn)
    ph = jnp.sum(h1n * onehot.astype(jnp.float32)[:, :, None], axis=1)
    npos_ref[...] = jnp.sum(tgt * ph, axis=1, keepdims=True) * INV_T

    psub = jnp.sum(h2[:, : S - 1, :], axis=1) * (1.0 / (S - 1))
    spos_ref[...] = jnp.sum(tgt * psub, axis=1, keepdims=True) * INV_T

    packed = jnp.concatenate(
        [h1n, jnp.broadcast_to(psub[:, None, :], (TB, S, NH))], axis=2)
    c_ref[...] = packed.reshape(TB * S, CW)


def _stage_a(bf2, bam, bfm2, ba, sn2, w_n, w_c, b_n2, b_c2, a_n2, a_c2):
    return pl.pallas_call(
        _stage_a_body,
        grid=(GRID,),
        in_specs=[
            pl.BlockSpec((TB * S, NIN), lambda i: (i, 0)),
            pl.BlockSpec((TB, S, S), lambda i: (i, 0, 0)),
            pl.BlockSpec((TB * S, NIN), lambda i: (i, 0)),
            pl.BlockSpec((TB, S, S), lambda i: (i, 0, 0)),
            pl.BlockSpec((TB, 1), lambda i: (i, 0)),
            pl.BlockSpec((NIN, NH), lambda i: (0, 0)),
            pl.BlockSpec((NIN, NH), lambda i: (0, 0)),
            pl.BlockSpec((1, NH), lambda i: (0, 0)),
            pl.BlockSpec((1, NH), lambda i: (0, 0)),
            pl.BlockSpec((1, 1), lambda i: (0, 0)),
            pl.BlockSpec((1, 1), lambda i: (0, 0)),
        ],
        out_specs=[
            pl.BlockSpec((TB * S, CW), lambda i: (i, 0)),
            pl.BlockSpec((TB, NH), lambda i: (i, 0)),
            pl.BlockSpec((TB, 1), lambda i: (i, 0)),
            pl.BlockSpec((TB, 1), lambda i: (i, 0)),
        ],
        out_shape=[
            jax.ShapeDtypeStruct((B * S, CW), jnp.float32),
            jax.ShapeDtypeStruct((B, NH), jnp.float32),
            jax.ShapeDtypeStruct((B, 1), jnp.float32),
            jax.ShapeDtypeStruct((B, 1), jnp.float32),
        ],
        compiler_params=pltpu.CompilerParams(
            dimension_semantics=("parallel",)),
    )(bf2, bam, bfm2, ba, sn2, w_n, w_c, b_n2, b_c2, a_n2, a_c2)


NC, NS = 2, 16
NW = NC * NS
CB = B // NW
M = 64
NSUB = CB // M
OUTW = 16


@functools.cache
def _build_stage_b():
  kern = functools.partial(
    pl.kernel,
    out_type=[jax.ShapeDtypeStruct((B, OUTW), jnp.float32),
              jax.ShapeDtypeStruct((B, OUTW), jnp.float32)],
    mesh=plsc.VectorSubcoreMesh(core_axis_name="c", subcore_axis_name="s",
                                num_cores=NC, num_subcores=NS),
    scratch_types=[
        pltpu.VMEM((M, NH), jnp.float32),
        pltpu.VMEM((M * K,), jnp.int32),
        pltpu.VMEM((M * K, CW), jnp.float32),
        pltpu.VMEM((M, OUTW), jnp.float32),
        pltpu.VMEM((M, OUTW), jnp.float32),
        pltpu.SemaphoreType.DMA,
    ],
  )

  @kern
  def _stage_b(c_hbm, tgt_hbm, idx_hbm, outn_hbm, outs_hbm,
               t_v, idx_v, rows_v, outn_v, outs_v, sem):
    wid = lax.axis_index("s") * NC + lax.axis_index("c")
    base_w = wid * CB

    def subchunk(sc, carry):
        base = base_w + sc * M
        pltpu.sync_copy(tgt_hbm.at[pl.ds(base, M)], t_v)
        pltpu.sync_copy(idx_hbm.at[pl.ds(base * K, M * K)], idx_v)
        pltpu.async_copy(c_hbm.at[idx_v], rows_v, sem).wait()

        def hsum(a):
            for st in (8, 4, 2, 1):
                a = a + jnp.take(a, lax.iota(jnp.int32, 16) ^ st)
            return a

        def dot_row(m, carry2):
            t0 = t_v[m, pl.ds(0, 16)]
            t1 = t_v[m, pl.ds(16, 16)]
            t2 = t_v[m, pl.ds(32, 16)]
            t3 = t_v[m, pl.ds(48, 16)]
            lane = lax.iota(jnp.int32, 16)
            res_n = jnp.zeros((16,), jnp.float32)
            res_s = jnp.zeros((16,), jnp.float32)
            for k in range(K):
                r = m * K + k
                a = (rows_v[r, pl.ds(0, 16)] * t0
                     + rows_v[r, pl.ds(16, 16)] * t1
                     + rows_v[r, pl.ds(32, 16)] * t2
                     + rows_v[r, pl.ds(48, 16)] * t3)
                res_n = jnp.where(lane == k, hsum(a) * INV_T, res_n)
                b = (rows_v[r, pl.ds(64, 16)] * t0
                     + rows_v[r, pl.ds(80, 16)] * t1
                     + rows_v[r, pl.ds(96, 16)] * t2
                     + rows_v[r, pl.ds(112, 16)] * t3)
                res_s = jnp.where(lane == k, hsum(b) * INV_T, res_s)
            outn_v[m, pl.ds(0, 16)] = res_n
            outs_v[m, pl.ds(0, 16)] = res_s
            return carry2

        lax.fori_loop(0, M, dot_row, 0)
        pltpu.sync_copy(outn_v, outn_hbm.at[pl.ds(base, M)])
        pltpu.sync_copy(outs_v, outs_hbm.at[pl.ds(base, M)])
        return carry

    lax.fori_loop(0, NSUB, subchunk, 0)

  return _stage_b


def kernel(bf_mask, ba, bf, ba_mask, multi_neg_node, sample_node,
           W_node, b_node, a_node, W_ctx, b_ctx, a_ctx):
    bf2 = bf.reshape(B * S, NIN)
    bfm2 = bf_mask.reshape(B * S, NIN)
    sn2 = sample_node.astype(jnp.int32).reshape(B, 1)
    b_n2 = b_node.reshape(1, NH)
    b_c2 = b_ctx.reshape(1, NH)
    a_n2 = a_node.reshape(1, 1)
    a_c2 = a_ctx.reshape(1, 1)

    ctab, tgt, npos, spos = _stage_a(
        bf2, ba_mask, bfm2, ba, sn2, W_node, W_ctx, b_n2, b_c2, a_n2, a_c2)

    mn = multi_neg_node.astype(jnp.int32)
    sn1 = sample_node.astype(jnp.int32)
    idx = (mn * S + sn1[None, :]).T.reshape(B * K)
    outn, outs = _build_stage_b()(ctab, tgt, idx)

    node_logits = jnp.concatenate([npos, outn[:, :K]], axis=1)
    sub_logits = jnp.concatenate([spos, outs[:, :K]], axis=1)
    labels = jnp.zeros((B,), dtype=jnp.int32)
    return (node_logits, sub_logits, labels)


# speedup vs baseline: 1.4241x; 1.4241x over previous
import functools

import jax
import jax.numpy as jnp
from jax import lax
from jax.experimental import pallas as pl
from jax.experimental.pallas import tpu as pltpu
from jax.experimental.pallas import tpu_sc as plsc

B, S, NIN, NH, K = 16384, 8, 128, 64, 10
INV_T = 5.0
CW = 2 * NH


TB = 512
GRID = B // TB


def _stage_a_body(bf_ref, bam_ref, bfm_ref, ba_ref,
                  wn_ref, wc_ref, bn_ref, bc_ref, an_ref, ac_ref,
                  c_ref, tgt_ref):
    x1 = jnp.dot(bf_ref[...], wn_ref[...],
                 preferred_element_type=jnp.float32).reshape(TB, S, NH)
    x2 = jnp.dot(bfm_ref[...], wc_ref[...],
                 preferred_element_type=jnp.float32).reshape(TB, S, NH)
    x12 = jnp.concatenate([x1, x2], axis=2)

    acat = jnp.concatenate([bam_ref[...], ba_ref[...]], axis=2)
    acat2 = acat.reshape(TB * S, 2 * S)
    col = lax.broadcasted_iota(jnp.int32, (2 * S, S * CW), 1)
    row = lax.broadcasted_iota(jnp.int32, (2 * S, S * CW), 0)
    jcol = col // CW
    hcol = col % CW
    emat = (((row == jcol) & (hcol < NH))
            | ((row == jcol + S) & (hcol >= NH))).astype(jnp.bfloat16)
    aexp = jnp.dot(acat2.astype(jnp.bfloat16), emat,
                   preferred_element_type=jnp.float32)

    acc = jnp.zeros((TB * S, CW), jnp.float32)
    for j in range(S):
        xb = jnp.broadcast_to(x12[:, j:j + 1, :], (TB, S, CW))
        acc = acc + aexp[:, j * CW:(j + 1) * CW] * xb.reshape(TB * S, CW)

    b12 = jnp.concatenate([bn_ref[...], bc_ref[...]], axis=1)
    lane = lax.broadcasted_iota(jnp.int32, (1, CW), 1)
    slope = jnp.where(lane < NH, an_ref[0, 0], ac_ref[0, 0])
    h12 = acc + b12
    h12 = jnp.where(h12 >= 0, h12, slope * h12)
    h12 = h12.reshape(TB, S, CW)

    h1 = h12[:, :, :NH]
    h2 = h12[:, :, NH:]
    ss = jnp.sum(h1 * h1, axis=2, keepdims=True)
    inv = lax.rsqrt(jnp.maximum(ss, 1e-24))
    h1n = h1 * inv
    tgt_ref[...] = h1n[:, S - 1, :]

    psub = jnp.sum(h2[:, : S - 1, :], axis=1) * (1.0 / (S - 1))

    packed = jnp.concatenate(
        [h1n, jnp.broadcast_to(psub[:, None, :], (TB, S, NH))], axis=2)
    c_ref[...] = packed.reshape(TB * S, CW)


def _stage_a(bf2, bam, bfm2, ba, w_n, w_c, b_n2, b_c2, a_n2, a_c2):
    return pl.pallas_call(
        _stage_a_body,
        grid=(GRID,),
        in_specs=[
            pl.BlockSpec((TB * S, NIN), lambda i: (i, 0)),
            pl.BlockSpec((TB, S, S), lambda i: (i, 0, 0)),
            pl.BlockSpec((TB * S, NIN), lambda i: (i, 0)),
            pl.BlockSpec((TB, S, S), lambda i: (i, 0, 0)),
            pl.BlockSpec((NIN, NH), lambda i: (0, 0)),
            pl.BlockSpec((NIN, NH), lambda i: (0, 0)),
            pl.BlockSpec((1, NH), lambda i: (0, 0)),
            pl.BlockSpec((1, NH), lambda i: (0, 0)),
            pl.BlockSpec((1, 1), lambda i: (0, 0)),
            pl.BlockSpec((1, 1), lambda i: (0, 0)),
        ],
        out_specs=[
            pl.BlockSpec((TB * S, CW), lambda i: (i, 0)),
            pl.BlockSpec((TB, NH), lambda i: (i, 0)),
        ],
        out_shape=[
            jax.ShapeDtypeStruct((B * S, CW), jnp.float32),
            jax.ShapeDtypeStruct((B, NH), jnp.float32),
        ],
        compiler_params=pltpu.CompilerParams(
            dimension_semantics=("parallel",)),
    )(bf2, bam, bfm2, ba, w_n, w_c, b_n2, b_c2, a_n2, a_c2)


NC, NS = 2, 16
NW = NC * NS
CB = B // NW
M = 64
NSUB = CB // M
KK = K + 1
OUTW = 16


@functools.cache
def _build_stage_b():
  kern = functools.partial(
    pl.kernel,
    out_type=[jax.ShapeDtypeStruct((B, OUTW), jnp.float32),
              jax.ShapeDtypeStruct((B, OUTW), jnp.float32)],
    mesh=plsc.VectorSubcoreMesh(core_axis_name="c", subcore_axis_name="s",
                                num_cores=NC, num_subcores=NS),
    scratch_types=[
        pltpu.VMEM((M, NH), jnp.float32),
        pltpu.VMEM((M * KK,), jnp.int32),
        pltpu.VMEM((M * KK, CW), jnp.float32),
        pltpu.VMEM((M, OUTW), jnp.float32),
        pltpu.VMEM((M, OUTW), jnp.float32),
        pltpu.SemaphoreType.DMA,
    ],
  )

  @kern
  def _stage_b(c_hbm, tgt_hbm, idx_hbm, outn_hbm, outs_hbm,
               t_v, idx_v, rows_v, outn_v, outs_v, sem):
    wid = lax.axis_index("s") * NC + lax.axis_index("c")
    base_w = wid * CB

    def subchunk(sc, carry):
        base = base_w + sc * M
        pltpu.sync_copy(tgt_hbm.at[pl.ds(base, M)], t_v)
        pltpu.sync_copy(idx_hbm.at[pl.ds(base * KK, M * KK)], idx_v)
        pltpu.async_copy(c_hbm.at[idx_v], rows_v, sem).wait()

        def hsum(a):
            for st in (8, 4, 2, 1):
                a = a + jnp.take(a, lax.iota(jnp.int32, 16) ^ st)
            return a

        def dot_row(m, carry2):
            t0 = t_v[m, pl.ds(0, 16)]
            t1 = t_v[m, pl.ds(16, 16)]
            t2 = t_v[m, pl.ds(32, 16)]
            t3 = t_v[m, pl.ds(48, 16)]
            lane = lax.iota(jnp.int32, 16)
            res_n = jnp.zeros((16,), jnp.float32)
            res_s = jnp.zeros((16,), jnp.float32)
            for k in range(KK):
                r = m * KK + k
                a = (rows_v[r, pl.ds(0, 16)] * t0
                     + rows_v[r, pl.ds(16, 16)] * t1
                     + rows_v[r, pl.ds(32, 16)] * t2
                     + rows_v[r, pl.ds(48, 16)] * t3)
                res_n = jnp.where(lane == k, hsum(a) * INV_T, res_n)
                b = (rows_v[r, pl.ds(64, 16)] * t0
                     + rows_v[r, pl.ds(80, 16)] * t1
                     + rows_v[r, pl.ds(96, 16)] * t2
                     + rows_v[r, pl.ds(112, 16)] * t3)
                res_s = jnp.where(lane == k, hsum(b) * INV_T, res_s)
            outn_v[m, pl.ds(0, 16)] = res_n
            outs_v[m, pl.ds(0, 16)] = res_s
            return carry2

        lax.fori_loop(0, M, dot_row, 0)
        pltpu.sync_copy(outn_v, outn_hbm.at[pl.ds(base, M)])
        pltpu.sync_copy(outs_v, outs_hbm.at[pl.ds(base, M)])
        return carry

    lax.fori_loop(0, NSUB, subchunk, 0)

  return _stage_b


def kernel(bf_mask, ba, bf, ba_mask, multi_neg_node, sample_node,
           W_node, b_node, a_node, W_ctx, b_ctx, a_ctx):
    bf2 = bf.reshape(B * S, NIN)
    bfm2 = bf_mask.reshape(B * S, NIN)
    b_n2 = b_node.reshape(1, NH)
    b_c2 = b_ctx.reshape(1, NH)
    a_n2 = a_node.reshape(1, 1)
    a_c2 = a_ctx.reshape(1, 1)

    ctab, tgt = _stage_a(
        bf2, ba_mask, bfm2, ba, W_node, W_ctx, b_n2, b_c2, a_n2, a_c2)

    mn = multi_neg_node.astype(jnp.int32)
    sn1 = sample_node.astype(jnp.int32)
    rows = mn * S + sn1[None, :]
    pos_row = (lax.iota(jnp.int32, B) * S + sn1)[None, :]
    idx = jnp.concatenate([rows, pos_row], axis=0).T.reshape(B * KK)
    outn, outs = _build_stage_b()(ctab, tgt, idx)

    node_logits = jnp.concatenate([outn[:, K:KK], outn[:, :K]], axis=1)
    sub_logits = jnp.concatenate([outs[:, K:KK], outs[:, :K]], axis=1)
    labels = jnp.zeros((B,), dtype=jnp.int32)
    return (node_logits, sub_logits, labels)


# speedup vs baseline: 1.6257x; 1.1416x over previous
import functools

import jax
import jax.numpy as jnp
from jax import lax
from jax.experimental import pallas as pl
from jax.experimental.pallas import tpu as pltpu
from jax.experimental.pallas import tpu_sc as plsc

B, S, NIN, NH, K = 16384, 8, 128, 64, 10
INV_T = 5.0
CW = 2 * NH


TB = 512
GRID = B // TB


def _stage_a_body(bf_ref, bam_ref, bfm_ref, ba_ref,
                  wn_ref, wc_ref, bn_ref, bc_ref, an_ref, ac_ref,
                  c_ref, tgt_ref):
    x1 = jnp.dot(bf_ref[...], wn_ref[...],
                 preferred_element_type=jnp.float32).reshape(TB, S, NH)
    x2 = jnp.dot(bfm_ref[...], wc_ref[...],
                 preferred_element_type=jnp.float32).reshape(TB, S, NH)
    x12 = jnp.concatenate([x1, x2], axis=2)

    acat = jnp.concatenate([bam_ref[...], ba_ref[...]], axis=2)
    acat2 = acat.reshape(TB * S, 2 * S)
    col = lax.broadcasted_iota(jnp.int32, (2 * S, S * CW), 1)
    row = lax.broadcasted_iota(jnp.int32, (2 * S, S * CW), 0)
    jcol = col // CW
    hcol = col % CW
    emat = (((row == jcol) & (hcol < NH))
            | ((row == jcol + S) & (hcol >= NH))).astype(jnp.bfloat16)
    aexp = jnp.dot(acat2.astype(jnp.bfloat16), emat,
                   preferred_element_type=jnp.float32)

    acc = jnp.zeros((TB * S, CW), jnp.float32)
    for j in range(S):
        xb = jnp.broadcast_to(x12[:, j:j + 1, :], (TB, S, CW))
        acc = acc + aexp[:, j * CW:(j + 1) * CW] * xb.reshape(TB * S, CW)

    b12 = jnp.concatenate([bn_ref[...], bc_ref[...]], axis=1)
    lane = lax.broadcasted_iota(jnp.int32, (1, CW), 1)
    slope = jnp.where(lane < NH, an_ref[0, 0], ac_ref[0, 0])
    h12 = acc + b12
    h12 = jnp.where(h12 >= 0, h12, slope * h12)
    h12 = h12.reshape(TB, S, CW)

    h1 = h12[:, :, :NH]
    h2 = h12[:, :, NH:]
    ss = jnp.sum(h1 * h1, axis=2, keepdims=True)
    inv = lax.rsqrt(jnp.maximum(ss, 1e-24))
    h1n = h1 * inv
    tgt_ref[...] = h1n[:, S - 1, :]

    psub = jnp.sum(h2[:, : S - 1, :], axis=1) * (1.0 / (S - 1))

    packed = jnp.concatenate(
        [h1n, jnp.broadcast_to(psub[:, None, :], (TB, S, NH))], axis=2)
    c_ref[...] = packed.reshape(TB * S, CW)


def _stage_a(bf2, bam, bfm2, ba, w_n, w_c, b_n2, b_c2, a_n2, a_c2):
    return pl.pallas_call(
        _stage_a_body,
        grid=(GRID,),
        in_specs=[
            pl.BlockSpec((TB * S, NIN), lambda i: (i, 0)),
            pl.BlockSpec((TB, S, S), lambda i: (i, 0, 0)),
            pl.BlockSpec((TB * S, NIN), lambda i: (i, 0)),
            pl.BlockSpec((TB, S, S), lambda i: (i, 0, 0)),
            pl.BlockSpec((NIN, NH), lambda i: (0, 0)),
            pl.BlockSpec((NIN, NH), lambda i: (0, 0)),
            pl.BlockSpec((1, NH), lambda i: (0, 0)),
            pl.BlockSpec((1, NH), lambda i: (0, 0)),
            pl.BlockSpec((1, 1), lambda i: (0, 0)),
            pl.BlockSpec((1, 1), lambda i: (0, 0)),
        ],
        out_specs=[
            pl.BlockSpec((TB * S, CW), lambda i: (i, 0)),
            pl.BlockSpec((TB, NH), lambda i: (i, 0)),
        ],
        out_shape=[
            jax.ShapeDtypeStruct((B * S, CW), jnp.float32),
            jax.ShapeDtypeStruct((B, NH), jnp.float32),
        ],
        compiler_params=pltpu.CompilerParams(
            dimension_semantics=("parallel",)),
    )(bf2, bam, bfm2, ba, w_n, w_c, b_n2, b_c2, a_n2, a_c2)


NC, NS = 2, 16
NW = NC * NS
CB = B // NW
M = 32
NSUB = CB // M
KK = K + 1
MK = M * KK
OUTW = 16


@functools.cache
def _build_stage_b():
  kern = functools.partial(
    pl.kernel,
    out_type=[jax.ShapeDtypeStruct((B, OUTW), jnp.float32),
              jax.ShapeDtypeStruct((B, OUTW), jnp.float32)],
    mesh=plsc.VectorSubcoreMesh(core_axis_name="c", subcore_axis_name="s",
                                num_cores=NC, num_subcores=NS),
    scratch_types=[
        pltpu.VMEM((CB * KK,), jnp.int32),
        pltpu.VMEM((2, M, NH), jnp.float32),
        pltpu.VMEM((2, MK, CW), jnp.float32),
        pltpu.VMEM((2, M, OUTW), jnp.float32),
        pltpu.VMEM((2, M, OUTW), jnp.float32),
        pltpu.SemaphoreType.DMA,
        pltpu.SemaphoreType.DMA,
        pltpu.SemaphoreType.DMA,
        pltpu.SemaphoreType.DMA,
        pltpu.SemaphoreType.DMA,
        pltpu.SemaphoreType.DMA,
        pltpu.SemaphoreType.DMA,
        pltpu.SemaphoreType.DMA,
    ],
  )

  @kern
  def _stage_b(c_hbm, tgt_hbm, idx_hbm, outn_hbm, outs_hbm,
               idx_all, t_v2, rows_v2, outn_v2, outs_v2,
               sg0, sg1, st0, st1, son0, son1, sos0, sos1):
    wid = lax.axis_index("s") * NC + lax.axis_index("c")
    base_w = wid * CB
    sem_g = (sg0, sg1)
    sem_t = (st0, st1)
    sem_on = (son0, son1)
    sem_os = (sos0, sos1)

    pltpu.sync_copy(idx_hbm.at[pl.ds(base_w * KK, CB * KK)], idx_all)

    def gather_cp(sc, d):
        src = c_hbm.at[idx_all.at[pl.ds(sc * MK, MK)]]
        return pltpu.make_async_copy(src, rows_v2.at[d], sem_g[d])

    def t_cp(sc, d):
        src = tgt_hbm.at[pl.ds(base_w + sc * M, M)]
        return pltpu.make_async_copy(src, t_v2.at[d], sem_t[d])

    def outn_cp(sc, d):
        dst = outn_hbm.at[pl.ds(base_w + sc * M, M)]
        return pltpu.make_async_copy(outn_v2.at[d], dst, sem_on[d])

    def outs_cp(sc, d):
        dst = outs_hbm.at[pl.ds(base_w + sc * M, M)]
        return pltpu.make_async_copy(outs_v2.at[d], dst, sem_os[d])

    def hsum(a):
        for st in (8, 4, 2, 1):
            a = a + jnp.take(a, lax.iota(jnp.int32, 16) ^ st)
        return a

    def compute(d):
        def dot_row(m, carry2):
            t0 = t_v2[d, m, pl.ds(0, 16)]
            t1 = t_v2[d, m, pl.ds(16, 16)]
            t2 = t_v2[d, m, pl.ds(32, 16)]
            t3 = t_v2[d, m, pl.ds(48, 16)]
            lane = lax.iota(jnp.int32, 16)
            res_n = jnp.zeros((16,), jnp.float32)
            res_s = jnp.zeros((16,), jnp.float32)
            for k in range(KK):
                r = m * KK + k
                a = (rows_v2[d, r, pl.ds(0, 16)] * t0
                     + rows_v2[d, r, pl.ds(16, 16)] * t1
                     + rows_v2[d, r, pl.ds(32, 16)] * t2
                     + rows_v2[d, r, pl.ds(48, 16)] * t3)
                res_n = jnp.where(lane == k, hsum(a) * INV_T, res_n)
                b = (rows_v2[d, r, pl.ds(64, 16)] * t0
                     + rows_v2[d, r, pl.ds(80, 16)] * t1
                     + rows_v2[d, r, pl.ds(96, 16)] * t2
                     + rows_v2[d, r, pl.ds(112, 16)] * t3)
                res_s = jnp.where(lane == k, hsum(b) * INV_T, res_s)
            outn_v2[d, m, pl.ds(0, 16)] = res_n
            outs_v2[d, m, pl.ds(0, 16)] = res_s
            return carry2

        lax.fori_loop(0, M, dot_row, 0)

    gather_cp(0, 0).start()
    t_cp(0, 0).start()
    gather_cp(1, 1).start()
    t_cp(1, 1).start()

    def pair(g, carry):
        for d in (0, 1):
            sc = g * 2 + d
            gather_cp(sc, d).wait()
            t_cp(sc, d).wait()

            @pl.when(sc >= 2)
            def _():
                outn_cp(sc - 2, d).wait()
                outs_cp(sc - 2, d).wait()

            compute(d)

            @pl.when(sc + 2 < NSUB)
            def _():
                gather_cp(sc + 2, d).start()
                t_cp(sc + 2, d).start()

            outn_cp(sc, d).start()
            outs_cp(sc, d).start()
        return carry

    lax.fori_loop(0, NSUB // 2, pair, 0)
    outn_cp(NSUB - 2, 0).wait()
    outs_cp(NSUB - 2, 0).wait()
    outn_cp(NSUB - 1, 1).wait()
    outs_cp(NSUB - 1, 1).wait()

  return _stage_b


def kernel(bf_mask, ba, bf, ba_mask, multi_neg_node, sample_node,
           W_node, b_node, a_node, W_ctx, b_ctx, a_ctx):
    bf2 = bf.reshape(B * S, NIN)
    bfm2 = bf_mask.reshape(B * S, NIN)
    b_n2 = b_node.reshape(1, NH)
    b_c2 = b_ctx.reshape(1, NH)
    a_n2 = a_node.reshape(1, 1)
    a_c2 = a_ctx.reshape(1, 1)

    ctab, tgt = _stage_a(
        bf2, ba_mask, bfm2, ba, W_node, W_ctx, b_n2, b_c2, a_n2, a_c2)

    mn = multi_neg_node.astype(jnp.int32)
    sn1 = sample_node.astype(jnp.int32)
    rows = mn * S + sn1[None, :]
    pos_row = (lax.iota(jnp.int32, B) * S + sn1)[None, :]
    idx = jnp.concatenate([rows, pos_row], axis=0).T.reshape(B * KK)
    outn, outs = _build_stage_b()(ctab, tgt, idx)

    node_logits = jnp.concatenate([outn[:, K:KK], outn[:, :K]], axis=1)
    sub_logits = jnp.concatenate([outs[:, K:KK], outs[:, :K]], axis=1)
    labels = jnp.zeros((B,), dtype=jnp.int32)
    return (node_logits, sub_logits, labels)
